# baseline (speedup 1.0000x reference)
import jax
import jax.numpy as jnp
from jax import lax
from jax.experimental import pallas as pl
from jax.experimental.pallas import tpu as pltpu
from jax.experimental.pallas import tpu_sc as plsc

_EMB = 128
_NFEAT = 7
_N = 100000
_OUT_D = _NFEAT * _EMB
_NC, _NS = 2, 16
_NW = _NC * _NS
_BIGW = 20
_NPW = 3128
_CNODES = 128
_NFULL = 24
_NCH = _NFULL + 1
_XSTRIDE = _NCH * _CNODES
_NUNIT = _NFULL * _NFEAT
_NPAIRU = _NUNIT // 2
_XTPAD = _NW * 8
_TOTAL_ROWS = 590
_OFFS = (0, 4, 261, 269, 326, 329, 332)


def _sc_body(tab_hbm, xt_hbm, out_hbm,
             tab_sp, xrows_v, idx_v, rows_a, rows_b, sem_a, sem_b):
    sid = lax.axis_index("s")
    wid = sid * _NC + lax.axis_index("c")

    @pl.when(sid == 0)
    def _():
        pltpu.sync_copy(tab_hbm, tab_sp)

    plsc.subcore_barrier()

    node0 = wid * _NPW - 8 * jnp.maximum(wid - _BIGW, 0)
    tail = jnp.where(wid < _BIGW, _NPW - _NFULL * _CNODES,
                     _NPW - 8 - _NFULL * _CNODES)

    for k in range(_NFEAT):
        pltpu.sync_copy(xt_hbm.at[pl.ds(k * _N + node0, _NPW)],
                        xrows_v.at[pl.ds(k * _XSTRIDE, _NPW)])

    def build_idx(c, carry):
        for k in range(_NFEAT):
            for h in range(8):
                x16 = xrows_v[pl.ds(k * _XSTRIDE + c * _CNODES + h * 16, 16)]
                idx_v[pl.ds((c * _NFEAT + k) * _EMB + h * 16, 16)] = jnp.clip(
                    x16 + jnp.int32(_OFFS[k]), 0, _TOTAL_ROWS - 1)
        return carry

    lax.fori_loop(0, _NCH, build_idx, None)

    bufs = (rows_a, rows_b)
    sems = (sem_a, sem_b)

    def g_half(v, half, b):
        p = lax.div(v, _NFEAT)
        k = lax.rem(v, _NFEAT)
        c = 2 * p + half
        return pltpu.make_async_copy(
            tab_sp.at[idx_v.at[pl.ds((c * _NFEAT + k) * _EMB, _EMB)]],
            bufs[b].at[pl.ds(half * _CNODES, _CNODES)], sems[b])

    def fire(v, b):
        g_half(v, 0, b).start()
        g_half(v, 1, b).start()

    def drain(v, b):
        p = lax.div(v, _NFEAT)
        k = lax.rem(v, _NFEAT)
        g_half(v, 0, b).wait()
        g_half(v, 1, b).wait()
        pltpu.sync_copy(
            bufs[b],
            out_hbm.at[pl.ds(node0 + p * 2 * _CNODES, 2 * _CNODES),
                       pl.ds(k * _EMB, _EMB)])

    fire(0, 0)
    fire(1, 1)

    def pipe(i, carry):
        va = 2 * i
        drain(va, 0)

        @pl.when(va + 2 < _NPAIRU)
        def _():
            fire(va + 2, 0)

        vb = va + 1
        drain(vb, 1)

        @pl.when(vb + 2 < _NPAIRU)
        def _():
            fire(vb + 2, 1)

        return carry

    lax.fori_loop(0, _NPAIRU // 2, pipe, None)

    def t_desc(k, b):
        return pltpu.make_async_copy(
            tab_sp.at[idx_v.at[pl.ds((_NFULL * _NFEAT + k) * _EMB, _EMB)]],
            bufs[b].at[pl.ds(0, _CNODES)], sems[b])

    def do_tail(n_out):
        t_desc(0, 0).start()
        for k in range(_NFEAT):
            b = k % 2
            if k + 1 < _NFEAT:
                t_desc(k + 1, 1 - b).start()
            t_desc(k, b).wait()
            pltpu.sync_copy(
                bufs[b].at[pl.ds(0, n_out)],
                out_hbm.at[pl.ds(node0 + _NFULL * _CNODES, n_out),
                           pl.ds(k * _EMB, _EMB)])

    @pl.when(tail == 56)
    def _():
        do_tail(56)

    @pl.when(tail == 48)
    def _():
        do_tail(48)


def kernel(x, W0, W1, W2, W3, W4, W5, W6):
    tab = jnp.concatenate([W0, W1, W2, W3, W4, W5, W6], axis=0)
    xt = jnp.pad(x.T.reshape(-1), (0, _XTPAD))
    run = pl.kernel(
        _sc_body,
        out_type=jax.ShapeDtypeStruct((_N, _OUT_D), jnp.float32),
        mesh=plsc.VectorSubcoreMesh(core_axis_name="c", subcore_axis_name="s"),
        scratch_types=[
            pltpu.VMEM_SHARED((_TOTAL_ROWS, _EMB), jnp.float32),
            pltpu.VMEM((_NFEAT * _XSTRIDE,), jnp.int32),
            pltpu.VMEM((_NCH * _NFEAT * _EMB,), jnp.int32),
            pltpu.VMEM((2 * _CNODES, _EMB), jnp.float32),
            pltpu.VMEM((2 * _CNODES, _EMB), jnp.float32),
            pltpu.SemaphoreType.DMA,
            pltpu.SemaphoreType.DMA,
        ],
    )
    return run(tab, xt)


# speedup vs baseline: 1.0316x; 1.0316x over previous
import jax
import jax.numpy as jnp
from jax import lax
from jax.experimental import pallas as pl
from jax.experimental.pallas import tpu as pltpu
from jax.experimental.pallas import tpu_sc as plsc

_EMB = 128
_NFEAT = 7
_N = 100000
_OUT_D = _NFEAT * _EMB
_NC, _NS = 2, 16
_NW = _NC * _NS
_BIGW = 20
_NPW = 3128
_CNODES = 128
_NFULL = 24
_NCH = _NFULL + 1
_XSTRIDE = _NCH * _CNODES
_NUNIT = _NFULL * _NFEAT
_XTPAD = _NW * 8
_TOTAL_ROWS = 590
_OFFS = (0, 4, 261, 269, 326, 329, 332)


def _sc_body(tab_hbm, xt_hbm, out_hbm,
             tab_sp, xrows_v, idx_v, rows_a, rows_b, sem_a, sem_b):
    sid = lax.axis_index("s")
    wid = sid * _NC + lax.axis_index("c")

    @pl.when(sid == 0)
    def _():
        pltpu.sync_copy(tab_hbm, tab_sp)

    plsc.subcore_barrier()

    node0 = wid * _NPW - 8 * jnp.maximum(wid - _BIGW, 0)
    tail = jnp.where(wid < _BIGW, _NPW - _NFULL * _CNODES,
                     _NPW - 8 - _NFULL * _CNODES)

    for k in range(_NFEAT):
        pltpu.sync_copy(xt_hbm.at[pl.ds(k * _N + node0, _NPW)],
                        xrows_v.at[pl.ds(k * _XSTRIDE, _NPW)])

    def build_idx(c, carry):
        for k in range(_NFEAT):
            for h in range(8):
                x16 = xrows_v[pl.ds(k * _XSTRIDE + c * _CNODES + h * 16, 16)]
                idx_v[pl.ds((c * _NFEAT + k) * _EMB + h * 16, 16)] = jnp.clip(
                    x16 + jnp.int32(_OFFS[k]), 0, _TOTAL_ROWS - 1)
        return carry

    lax.fori_loop(0, _NCH, build_idx, None)

    bufs = (rows_a, rows_b)
    sems = (sem_a, sem_b)

    def g_desc(u, b):
        return pltpu.make_async_copy(
            tab_sp.at[idx_v.at[pl.ds(u * _EMB, _EMB)]], bufs[b], sems[b])

    def out_stream(u, b, n_out):
        c = lax.div(u, _NFEAT)
        k = lax.rem(u, _NFEAT)
        pltpu.sync_copy(
            bufs[b].at[pl.ds(0, n_out)],
            out_hbm.at[pl.ds(node0 + c * _CNODES, n_out),
                       pl.ds(k * _EMB, _EMB)])

    g_desc(0, 0).start()
    g_desc(1, 1).start()

    def pipe(i, carry):
        ua = 2 * i
        g_desc(ua, 0).wait()
        out_stream(ua, 0, _CNODES)

        @pl.when(ua + 2 < _NUNIT)
        def _():
            g_desc(ua + 2, 0).start()

        ub = ua + 1
        g_desc(ub, 1).wait()
        out_stream(ub, 1, _CNODES)

        @pl.when(ub + 2 < _NUNIT)
        def _():
            g_desc(ub + 2, 1).start()

        return carry

    lax.fori_loop(0, _NUNIT // 2, pipe, None)

    def do_tail(n_out):
        g_desc(_NUNIT, 0).start()
        for k in range(_NFEAT):
            b = k % 2
            if k + 1 < _NFEAT:
                g_desc(_NUNIT + k + 1, 1 - b).start()
            g_desc(_NUNIT + k, b).wait()
            out_stream(_NUNIT + k, b, n_out)

    @pl.when(tail == 56)
    def _():
        do_tail(56)

    @pl.when(tail == 48)
    def _():
        do_tail(48)


def kernel(x, W0, W1, W2, W3, W4, W5, W6):
    tab = jnp.concatenate([W0, W1, W2, W3, W4, W5, W6], axis=0)
    xt = jnp.pad(x.T.reshape(-1), (0, _XTPAD))
    run = pl.kernel(
        _sc_body,
        out_type=jax.ShapeDtypeStruct((_N, _OUT_D), jnp.float32),
        mesh=plsc.VectorSubcoreMesh(core_axis_name="c", subcore_axis_name="s"),
        scratch_types=[
            pltpu.VMEM_SHARED((_TOTAL_ROWS, _EMB), jnp.float32),
            pltpu.VMEM((_NFEAT * _XSTRIDE,), jnp.int32),
            pltpu.VMEM((_NCH * _NFEAT * _EMB,), jnp.int32),
            pltpu.VMEM((_CNODES, _EMB), jnp.float32),
            pltpu.VMEM((_CNODES, _EMB), jnp.float32),
            pltpu.SemaphoreType.DMA,
            pltpu.SemaphoreType.DMA,
        ],
    )
    return run(tab, xt)


# speedup vs baseline: 1.0510x; 1.0188x over previous
import jax
import jax.numpy as jnp
from jax import lax
from jax.experimental import pallas as pl
from jax.experimental.pallas import tpu as pltpu
from jax.experimental.pallas import tpu_sc as plsc

_EMB = 128
_NFEAT = 7
_N = 100000
_OUT_D = _NFEAT * _EMB
_NC, _NS = 2, 16
_NW = _NC * _NS
_BIGW = 20
_NPW = 3128
_CNODES = 128
_NFULL = 24
_NCH = _NFULL + 1
_XSTRIDE = _NCH * _CNODES
_NUNIT = _NFULL * _NFEAT
_XTPAD = _NW * 8
_TOTAL_ROWS = 590
_OFFS = (0, 4, 261, 269, 326, 329, 332)


def _sc_body(tab_hbm, xt_hbm, out_hbm,
             tab_sp, xrows_v, idx_v, rows_a, rows_b, sem_a, sem_b):
    sid = lax.axis_index("s")
    wid = sid * _NC + lax.axis_index("c")

    node0 = wid * _NPW - 8 * jnp.maximum(wid - _BIGW, 0)
    tail = jnp.where(wid < _BIGW, _NPW - _NFULL * _CNODES,
                     _NPW - 8 - _NFULL * _CNODES)

    def x_desc(k):
        return pltpu.make_async_copy(
            xt_hbm.at[pl.ds(k * _N + node0, _NPW)],
            xrows_v.at[pl.ds(k * _XSTRIDE, _NPW)], sem_a)

    for k in range(_NFEAT):
        x_desc(k).start()

    @pl.when(sid == 0)
    def _():
        pltpu.sync_copy(tab_hbm, tab_sp)

    for k in range(_NFEAT):
        x_desc(k).wait()

    def build_idx(c, carry):
        for k in range(_NFEAT):
            for h in range(8):
                x16 = xrows_v[pl.ds(k * _XSTRIDE + c * _CNODES + h * 16, 16)]
                idx_v[pl.ds((c * _NFEAT + k) * _EMB + h * 16, 16)] = jnp.clip(
                    x16 + jnp.int32(_OFFS[k]), 0, _TOTAL_ROWS - 1)
        return carry

    lax.fori_loop(0, _NCH, build_idx, None)

    plsc.subcore_barrier()

    bufs = (rows_a, rows_b)
    sems = (sem_a, sem_b)

    def g_desc(u, b):
        return pltpu.make_async_copy(
            tab_sp.at[idx_v.at[pl.ds(u * _EMB, _EMB)]], bufs[b], sems[b])

    def out_stream(u, b, n_out):
        c = lax.div(u, _NFEAT)
        k = lax.rem(u, _NFEAT)
        pltpu.sync_copy(
            bufs[b].at[pl.ds(0, n_out)],
            out_hbm.at[pl.ds(node0 + c * _CNODES, n_out),
                       pl.ds(k * _EMB, _EMB)])

    g_desc(0, 0).start()
    g_desc(1, 1).start()

    def pipe(i, carry):
        ua = 2 * i
        g_desc(ua, 0).wait()
        out_stream(ua, 0, _CNODES)

        @pl.when(ua + 2 < _NUNIT)
        def _():
            g_desc(ua + 2, 0).start()

        ub = ua + 1
        g_desc(ub, 1).wait()
        out_stream(ub, 1, _CNODES)

        @pl.when(ub + 2 < _NUNIT)
        def _():
            g_desc(ub + 2, 1).start()

        return carry

    lax.fori_loop(0, _NUNIT // 2, pipe, None)

    def do_tail(n_out):
        g_desc(_NUNIT, 0).start()
        for k in range(_NFEAT):
            b = k % 2
            if k + 1 < _NFEAT:
                g_desc(_NUNIT + k + 1, 1 - b).start()
            g_desc(_NUNIT + k, b).wait()
            out_stream(_NUNIT + k, b, n_out)

    @pl.when(tail == 56)
    def _():
        do_tail(56)

    @pl.when(tail == 48)
    def _():
        do_tail(48)


def kernel(x, W0, W1, W2, W3, W4, W5, W6):
    tab = jnp.concatenate([W0, W1, W2, W3, W4, W5, W6], axis=0)
    xt = jnp.pad(x.T.reshape(-1), (0, _XTPAD))
    run = pl.kernel(
        _sc_body,
        out_type=jax.ShapeDtypeStruct((_N, _OUT_D), jnp.float32),
        mesh=plsc.VectorSubcoreMesh(core_axis_name="c", subcore_axis_name="s"),
        scratch_types=[
            pltpu.VMEM_SHARED((_TOTAL_ROWS, _EMB), jnp.float32),
            pltpu.VMEM((_NFEAT * _XSTRIDE,), jnp.int32),
            pltpu.VMEM((_NCH * _NFEAT * _EMB,), jnp.int32),
            pltpu.VMEM((_CNODES, _EMB), jnp.float32),
            pltpu.VMEM((_CNODES, _EMB), jnp.float32),
            pltpu.SemaphoreType.DMA,
            pltpu.SemaphoreType.DMA,
        ],
    )
    return run(tab, xt)


# speedup vs baseline: 1.0892x; 1.0364x over previous
import jax
import jax.numpy as jnp
from jax import lax
from jax.experimental import pallas as pl
from jax.experimental.pallas import tpu as pltpu
from jax.experimental.pallas import tpu_sc as plsc

_EMB = 128
_NFEAT = 7
_N = 100000
_OUT_D = _NFEAT * _EMB
_NC, _NS = 2, 16
_NW = _NC * _NS
_BIGW = 20
_NPW = 3128
_CNODES = 128
_NFULL = 24
_NCH = _NFULL + 1
_XSTRIDE = _NCH * _CNODES
_NUNIT = _NFULL * _NFEAT
_XTPAD = _NW * 8
_TOTAL_ROWS = 590
_OFFS = (0, 4, 261, 269, 326, 329, 332)


def _sc_body(tab_hbm, xt_hbm, out_hbm,
             tab_sp, xrows_v, idx_v, rows_a, rows_b, sem_a, sem_b):
    sid = lax.axis_index("s")
    wid = sid * _NC + lax.axis_index("c")

    node0 = wid * _NPW - 8 * jnp.maximum(wid - _BIGW, 0)
    tail = jnp.where(wid < _BIGW, _NPW - _NFULL * _CNODES,
                     _NPW - 8 - _NFULL * _CNODES)

    def x_desc(k):
        return pltpu.make_async_copy(
            xt_hbm.at[pl.ds(k * _N + node0, _NPW)],
            xrows_v.at[pl.ds(k * _XSTRIDE, _NPW)], sem_a)

    for k in range(_NFEAT):
        x_desc(k).start()

    @pl.when(sid == 0)
    def _():
        pltpu.sync_copy(tab_hbm, tab_sp)

    for k in range(_NFEAT):
        x_desc(k).wait()

    def build_idx(c):
        for k in range(_NFEAT):
            for h in range(8):
                x16 = xrows_v[pl.ds(k * _XSTRIDE + c * _CNODES + h * 16, 16)]
                idx_v[pl.ds((c * _NFEAT + k) * _EMB + h * 16, 16)] = jnp.clip(
                    x16 + jnp.int32(_OFFS[k]), 0, _TOTAL_ROWS - 1)

    build_idx(0)
    build_idx(1)

    plsc.subcore_barrier()

    bufs = (rows_a, rows_b)
    sems = (sem_a, sem_b)

    def g_desc(u, b):
        return pltpu.make_async_copy(
            tab_sp.at[idx_v.at[pl.ds(u * _EMB, _EMB)]], bufs[b], sems[b])

    def out_stream(u, b, n_out):
        c = lax.div(u, _NFEAT)
        k = lax.rem(u, _NFEAT)
        pltpu.sync_copy(
            bufs[b].at[pl.ds(0, n_out)],
            out_hbm.at[pl.ds(node0 + c * _CNODES, n_out),
                       pl.ds(k * _EMB, _EMB)])

    g_desc(0, 0).start()
    g_desc(1, 1).start()

    def process(u, b):
        g_desc(u, b).wait()
        out_stream(u, b, _CNODES)

        @pl.when(lax.rem(u, _NFEAT) == 0)
        def _():
            c2 = lax.div(u, _NFEAT) + 2

            @pl.when(c2 < _NCH)
            def _():
                build_idx(c2)

        @pl.when(u + 2 < _NUNIT)
        def _():
            g_desc(u + 2, b).start()

    def pipe(i, carry):
        process(2 * i, 0)
        process(2 * i + 1, 1)
        return carry

    lax.fori_loop(0, _NUNIT // 2, pipe, None)

    def do_tail(n_out):
        g_desc(_NUNIT, 0).start()
        for k in range(_NFEAT):
            b = k % 2
            if k + 1 < _NFEAT:
                g_desc(_NUNIT + k + 1, 1 - b).start()
            g_desc(_NUNIT + k, b).wait()
            out_stream(_NUNIT + k, b, n_out)

    @pl.when(tail == 56)
    def _():
        do_tail(56)

    @pl.when(tail == 48)
    def _():
        do_tail(48)


def kernel(x, W0, W1, W2, W3, W4, W5, W6):
    tab = jnp.concatenate([W0, W1, W2, W3, W4, W5, W6], axis=0)
    xt = jnp.pad(x.T.reshape(-1), (0, _XTPAD))
    run = pl.kernel(
        _sc_body,
        out_type=jax.ShapeDtypeStruct((_N, _OUT_D), jnp.float32),
        mesh=plsc.VectorSubcoreMesh(core_axis_name="c", subcore_axis_name="s"),
        scratch_types=[
            pltpu.VMEM_SHARED((_TOTAL_ROWS, _EMB), jnp.float32),
            pltpu.VMEM((_NFEAT * _XSTRIDE,), jnp.int32),
            pltpu.VMEM((_NCH * _NFEAT * _EMB,), jnp.int32),
            pltpu.VMEM((_CNODES, _EMB), jnp.float32),
            pltpu.VMEM((_CNODES, _EMB), jnp.float32),
            pltpu.SemaphoreType.DMA,
            pltpu.SemaphoreType.DMA,
        ],
    )
    return run(tab, xt)
